# Initial kernel scaffold; baseline (speedup 1.0000x reference)
import jax
import jax.numpy as jnp
from jax.experimental import pallas as pl


def kernel(x, target):
    raise NotImplementedError("write your pallas kernel here")


# speedup vs baseline: 48.4175x; 48.4175x over previous
import functools
import math

import jax
import jax.numpy as jnp
from jax import lax
from jax.experimental import pallas as pl
from jax.experimental.pallas import tpu as pltpu
from jax.experimental.pallas import tpu_sc as plsc

SMOOTH = 0.1
V = 1000
N = 8192

NC, NS, L = 2, 16, 16
NW = NC * NS
ROWS_PER_W = N // NW
CH_ROWS = 8
N_CH = ROWS_PER_W // CH_ROWS
CH_ELEMS = CH_ROWS * V
NFULL = V // L
TAIL = V - NFULL * L

MASK_WORDS = 1024


def _sc_partials(tgt_flat, x_flat):
    mesh = plsc.VectorSubcoreMesh(
        core_axis_name="c", subcore_axis_name="s",
        num_cores=NC, num_subcores=NS)

    @functools.partial(
        pl.kernel,
        out_type=jax.ShapeDtypeStruct((NW, 48), jnp.float32),
        mesh=mesh,
        scratch_types=[
            pltpu.VMEM((CH_ELEMS + 2 * L,), jnp.int32),
            pltpu.VMEM((CH_ELEMS + 2 * L,), jnp.float32),
            pltpu.VMEM((MASK_WORDS,), jnp.int32),
            pltpu.VMEM((48,), jnp.float32),
        ],
        compiler_params=pltpu.CompilerParams(needs_layout_passes=False),
    )
    def k(tgt_hbm, x_hbm, out_hbm, idx_buf, x_buf, mask_buf, acc_buf):
        i32 = jnp.int32
        wid = lax.axis_index("s") * i32(NC) + lax.axis_index("c")

        lanes = lax.broadcasted_iota(jnp.int32, (L,), 0)
        tail_mask = lanes < TAIL
        zero_i = jnp.zeros((L,), jnp.int32)
        zero_f = jnp.zeros((L,), jnp.float32)
        one_f = jnp.ones((L,), jnp.float32)

        def zero_body(j, carry):
            mask_buf[pl.ds(j * i32(L), L)] = zero_i
            return carry

        lax.fori_loop(jnp.int32(0), jnp.int32(MASK_WORDS // L), zero_body, jnp.int32(0))

        def chunk_body(c, accs):
            row0 = wid * i32(ROWS_PER_W) + c * i32(CH_ROWS)
            start = row0 * i32(V)
            pltpu.sync_copy(tgt_hbm.at[pl.ds(start, CH_ELEMS)],
                            idx_buf.at[pl.ds(0, CH_ELEMS)])
            pltpu.sync_copy(x_hbm.at[pl.ds(start, CH_ELEMS)],
                            x_buf.at[pl.ds(0, CH_ELEMS)])

            def row_body(r, accs):
                base = r * i32(V)
                epoch = row0 + r + i32(1)
                epoch_vec = zero_i + epoch

                def scat_body(j, carry):
                    idxv = idx_buf[pl.ds(base + j * i32(L), L)]
                    plsc.store_scatter(mask_buf, [idxv], epoch_vec)
                    return carry

                lax.fori_loop(jnp.int32(0), jnp.int32(NFULL), scat_body, jnp.int32(0))
                idxv = idx_buf[pl.ds(base + i32(NFULL * L), L)]
                plsc.store_scatter(mask_buf, [idxv], epoch_vec,
                                   mask=tail_mask)

                def red_body(j, accs):
                    acc_cnt, acc_hit, acc_all = accs
                    m = mask_buf[pl.ds(j * i32(L), L)]
                    xv = x_buf[pl.ds(base + j * i32(L), L)]
                    hit = m == epoch_vec
                    acc_cnt = acc_cnt + jnp.where(hit, one_f, zero_f)
                    acc_hit = acc_hit + jnp.where(hit, xv, zero_f)
                    acc_all = acc_all + xv
                    return (acc_cnt, acc_hit, acc_all)

                accs = lax.fori_loop(jnp.int32(0), jnp.int32(NFULL), red_body, accs)
                acc_cnt, acc_hit, acc_all = accs
                m = mask_buf[pl.ds(NFULL * L, L)]
                xv = x_buf[pl.ds(base + i32(NFULL * L), L)]
                hit = (m == epoch_vec) & tail_mask
                xsel = jnp.where(tail_mask, xv, zero_f)
                acc_cnt = acc_cnt + jnp.where(hit, one_f, zero_f)
                acc_hit = acc_hit + jnp.where(hit, xsel, zero_f)
                acc_all = acc_all + xsel
                return (acc_cnt, acc_hit, acc_all)

            return lax.fori_loop(jnp.int32(0), jnp.int32(CH_ROWS), row_body, accs)

        acc_cnt, acc_hit, acc_all = lax.fori_loop(
            jnp.int32(0), jnp.int32(N_CH), chunk_body, (zero_f, zero_f, zero_f))

        acc_buf[pl.ds(0, L)] = acc_cnt
        acc_buf[pl.ds(L, L)] = acc_hit
        acc_buf[pl.ds(2 * L, L)] = acc_all
        pltpu.sync_copy(acc_buf, out_hbm.at[wid])

    return k(tgt_flat, x_flat)


def kernel(x, target):
    tgt_flat = target.astype(jnp.int32).reshape(-1)
    x_flat = x.reshape(-1)
    parts = _sc_partials(tgt_flat, x_flat)
    cnt = jnp.sum(parts[:, 0:16])
    s_hit = jnp.sum(parts[:, 16:32])
    s_all = jnp.sum(parts[:, 32:48])
    w = SMOOTH / V
    q = 1.0 - SMOOTH
    lw = math.log(w)
    lq = math.log(q)
    loss = (jnp.float32(N * V * w * lw)
            - jnp.float32(w) * s_all
            + cnt * jnp.float32(q * lq - w * lw)
            - jnp.float32(q - w) * s_hit)
    return loss.astype(jnp.float32)
